# baseline (speedup 1.0000x reference)
import functools

import jax
import jax.numpy as jnp
from jax import lax
from jax.experimental import pallas as pl
from jax.experimental.pallas import tpu as pltpu
from jax.experimental.pallas import tpu_sc as plsc

VOCAB = 100000
WINDOW = 5000
BSZ = 16
BEAM = 8
K = 8
HALF = BEAM // 2
LANES = 16
NVEC = WINDOW // LANES
NEG_INF = float("-inf")
I32_BIG = 2**31 - 1


def _insert_topk(R, RI, x, xi):
    for lvl in range(K):
        swap = x > R[lvl]
        R[lvl], x = jnp.where(swap, x, R[lvl]), jnp.where(swap, R[lvl], x)
        RI[lvl], xi = jnp.where(swap, xi, RI[lvl]), jnp.where(swap, RI[lvl], xi)
    return R, RI


def _sc_body(starts_hbm, bias_hbm, lprobs_hbm, out_s, out_t, out_b,
             starts_v, bias_v, lbuf, cand_v, cand_i, cand2_v, cand2_i,
             rowf, rowt, rowb, sh_v, sh_i, sem):
    c = lax.axis_index("c")
    s = lax.axis_index("s")
    batch = c * 8 + s // 2
    half = s % 2

    pltpu.sync_copy(starts_hbm, starts_v)
    pltpu.sync_copy(bias_hbm, bias_v)
    start = pl.multiple_of(starts_v[pl.ds(batch * LANES, LANES)][0], 8)

    cps = []
    for j in range(HALF):
        jg = half * HALF + j
        cps.append(pltpu.async_copy(
            lprobs_hbm.at[batch, jg, pl.ds(start, WINDOW)], lbuf.at[j], sem))
    for cp in cps:
        cp.wait()

    iota = lax.iota(jnp.int32, LANES)
    neg = jnp.full((LANES,), NEG_INF, jnp.float32)

    carry = tuple([neg] * K + [jnp.zeros((LANES,), jnp.int32)] * K)
    for j in range(HALF):
        jg = half * HALF + j
        bias = bias_v[pl.ds((batch * BEAM + jg) * LANES, LANES)]
        base = jg * VOCAB + start

        def body(i, car, _j=j, _bias=bias, _base=base):
            x = lbuf[_j, pl.ds(i * LANES, LANES)] + _bias
            hit = plsc.all_reduce_population_count(x > car[K - 1])[0] > 0

            def do(car):
                R = list(car[:K])
                RI = list(car[K:])
                xi = jnp.full((LANES,), _base + i * LANES, jnp.int32) + iota
                R, RI = _insert_topk(R, RI, x, xi)
                return tuple(R + RI)

            return lax.cond(hit, do, lambda car: car, car)

        carry = lax.fori_loop(0, NVEC, body, carry)
        R = list(carry[:K])
        RI = list(carry[K:])
        x = lbuf[j, pl.ds(WINDOW - LANES, LANES)] + bias
        xi = jnp.full((LANES,), base + WINDOW - LANES, jnp.int32) + iota
        R, RI = _insert_topk(R, RI, x, xi)
        carry = tuple(R + RI)

    R = list(carry[:K])
    RI = list(carry[K:])
    for lvl in range(K):
        cand_v[lvl, :] = R[lvl]
        cand_i[lvl, :] = RI[lvl]

    pltpu.sync_copy(cand_v, sh_v.at[s])
    pltpu.sync_copy(cand_i, sh_i.at[s])
    plsc.subcore_barrier()

    @pl.when(half == 0)
    def _merge():
        pltpu.sync_copy(sh_v.at[s + 1], cand2_v)
        pltpu.sync_copy(sh_i.at[s + 1], cand2_i)
        vs = [cand_v[l, :] for l in range(K)] + [cand2_v[l, :] for l in range(K)]
        ids = [cand_i[l, :] for l in range(K)] + [cand2_i[l, :] for l in range(K)]
        accv = neg
        acci = jnp.zeros((LANES,), jnp.int32)
        for r in range(K):
            m = vs[0]
            for v in vs[1:]:
                m = jnp.maximum(m, v)
            gmax = jnp.full((LANES,), jnp.max(m))
            big = jnp.full((LANES,), I32_BIG, jnp.int32)
            cand = [jnp.where(vs[t] == gmax, ids[t], big) for t in range(2 * K)]
            mn = cand[0]
            for cnd in cand[1:]:
                mn = jnp.minimum(mn, cnd)
            gidx = jnp.full((LANES,), jnp.min(mn))
            vs = [jnp.where((vs[t] == gmax) & (ids[t] == gidx), neg, vs[t])
                  for t in range(2 * K)]
            accv = jnp.where(iota == r, gmax, accv)
            acci = jnp.where(iota == r, gidx, acci)
        beams = jnp.zeros((LANES,), jnp.int32)
        for t in range(1, BEAM):
            beams = beams + jnp.where(acci >= t * VOCAB, 1, 0)
        toks = acci - beams * VOCAB
        rowf[...] = accv
        rowt[...] = toks
        rowb[...] = beams
        pltpu.sync_copy(rowf.at[pl.ds(0, K)], out_s.at[batch])
        pltpu.sync_copy(rowt.at[pl.ds(0, K)], out_t.at[batch])
        pltpu.sync_copy(rowb.at[pl.ds(0, K)], out_b.at[batch])


_sc_call = functools.partial(
    pl.kernel,
    out_type=[
        jax.ShapeDtypeStruct((BSZ, BEAM), jnp.float32),
        jax.ShapeDtypeStruct((BSZ, BEAM), jnp.int32),
        jax.ShapeDtypeStruct((BSZ, BEAM), jnp.int32),
    ],
    mesh=plsc.VectorSubcoreMesh(core_axis_name="c", subcore_axis_name="s"),
    scratch_types=[
        pltpu.VMEM((BSZ * LANES,), jnp.int32),
        pltpu.VMEM((BSZ * BEAM * LANES,), jnp.float32),
        pltpu.VMEM((HALF, WINDOW), jnp.float32),
        pltpu.VMEM((K, LANES), jnp.float32),
        pltpu.VMEM((K, LANES), jnp.int32),
        pltpu.VMEM((K, LANES), jnp.float32),
        pltpu.VMEM((K, LANES), jnp.int32),
        pltpu.VMEM((LANES,), jnp.float32),
        pltpu.VMEM((LANES,), jnp.int32),
        pltpu.VMEM((LANES,), jnp.int32),
        pltpu.VMEM_SHARED((LANES, K, LANES), jnp.float32),
        pltpu.VMEM_SHARED((LANES, K, LANES), jnp.int32),
        pltpu.SemaphoreType.DMA,
    ],
    compiler_params=pltpu.CompilerParams(
        use_tc_tiling_on_sc=False, needs_layout_passes=False),
)(_sc_body)


def kernel(step, lprobs, scores, prev_output_tokens, original_batch_idxs):
    starts = (original_batch_idxs.astype(jnp.int32) * 1000) % VOCAB
    starts = jnp.repeat(starts, LANES)
    bias = jnp.repeat(jnp.take(scores, step - 1, axis=2).reshape(-1), LANES)
    scores_buf, indices_buf, beams_buf = _sc_call(starts, bias, lprobs)
    return scores_buf, indices_buf, beams_buf


# speedup vs baseline: 1.1861x; 1.1861x over previous
import functools

import jax
import jax.numpy as jnp
from jax import lax
from jax.experimental import pallas as pl
from jax.experimental.pallas import tpu as pltpu
from jax.experimental.pallas import tpu_sc as plsc

VOCAB = 100000
WINDOW = 5000
BSZ = 16
BEAM = 8
K = 8
HALF = BEAM // 2
LANES = 16
NVEC = WINDOW // LANES
BLK = 8
BLKW = BLK * LANES
NBLK = NVEC // BLK
NEG_INF = float("-inf")
I32_BIG = 2**31 - 1


def _insert_topk(R, RI, x, xi):
    for lvl in range(K):
        swap = x > R[lvl]
        R[lvl], x = jnp.where(swap, x, R[lvl]), jnp.where(swap, R[lvl], x)
        RI[lvl], xi = jnp.where(swap, xi, RI[lvl]), jnp.where(swap, RI[lvl], xi)
    return R, RI


def _sc_body(starts_hbm, bias_hbm, lprobs_hbm, out_s, out_t, out_b,
             starts_v, bias_v, lbuf, blkmax_v, cand_v, cand_i, cand2_v,
             cand2_i, rowf, rowt, rowb, sh_v, sh_i, sem):
    c = lax.axis_index("c")
    s = lax.axis_index("s")
    batch = c * 8 + s // 2
    half = s % 2

    pltpu.sync_copy(starts_hbm, starts_v)
    pltpu.sync_copy(bias_hbm, bias_v)
    start = pl.multiple_of(starts_v[pl.ds(batch * LANES, LANES)][0], 8)

    cps = []
    for j in range(HALF):
        jg = half * HALF + j
        cps.append(pltpu.async_copy(
            lprobs_hbm.at[batch, jg, pl.ds(start, WINDOW)], lbuf.at[j], sem))
    for cp in cps:
        cp.wait()

    iota = lax.iota(jnp.int32, LANES)
    neg = jnp.full((LANES,), NEG_INF, jnp.float32)

    biases = []
    M = neg
    for j in range(HALF):
        jg = half * HALF + j
        bias = bias_v[pl.ds((batch * BEAM + jg) * LANES, LANES)]
        biases.append(bias)

        def bodyA(b, M, _j=j, _bias=bias):
            m = lbuf[_j, pl.ds(b * BLKW, LANES)]
            for u in range(1, BLK):
                m = jnp.maximum(m, lbuf[_j, pl.ds(b * BLKW + u * LANES, LANES)])
            m = m + _bias
            blkmax_v[pl.ds((_j * (NBLK + 1) + b) * LANES, LANES)] = m
            return jnp.maximum(M, m)

        M = lax.fori_loop(0, NBLK, bodyA, M)
        m = lbuf[j, pl.ds(WINDOW - LANES, LANES)] + bias
        M = jnp.maximum(M, m)

    ms, _ = plsc.sort_key_val(M, M)
    t = jnp.full((LANES,), ms[LANES - K])

    carry = tuple([neg] * K + [jnp.zeros((LANES,), jnp.int32)] * K)
    for j in range(HALF):
        jg = half * HALF + j
        bias = biases[j]
        base = jg * VOCAB + start

        def bodyB(b, car, _j=j, _bias=bias, _base=base):
            bm = blkmax_v[pl.ds((_j * (NBLK + 1) + b) * LANES, LANES)]
            hit = plsc.all_reduce_population_count(bm >= t)[0] > 0

            def do(car):
                R = list(car[:K])
                RI = list(car[K:])
                for u in range(BLK):
                    x = lbuf[_j, pl.ds(b * BLKW + u * LANES, LANES)] + _bias
                    xi = jnp.full((LANES,), _base + b * BLKW + u * LANES,
                                  jnp.int32) + iota
                    R, RI = _insert_topk(R, RI, x, xi)
                return tuple(R + RI)

            return lax.cond(hit, do, lambda car: car, car)

        carry = lax.fori_loop(0, NBLK, bodyB, carry)
        R = list(carry[:K])
        RI = list(carry[K:])
        x = lbuf[j, pl.ds(WINDOW - LANES, LANES)] + bias
        xi = jnp.full((LANES,), base + WINDOW - LANES, jnp.int32) + iota
        R, RI = _insert_topk(R, RI, x, xi)
        carry = tuple(R + RI)

    R = list(carry[:K])
    RI = list(carry[K:])
    for lvl in range(K):
        cand_v[lvl, :] = R[lvl]
        cand_i[lvl, :] = RI[lvl]

    pltpu.sync_copy(cand_v, sh_v.at[s])
    pltpu.sync_copy(cand_i, sh_i.at[s])
    plsc.subcore_barrier()

    @pl.when(half == 0)
    def _merge():
        pltpu.sync_copy(sh_v.at[s + 1], cand2_v)
        pltpu.sync_copy(sh_i.at[s + 1], cand2_i)
        vs = [cand_v[l, :] for l in range(K)] + [cand2_v[l, :] for l in range(K)]
        ids = [cand_i[l, :] for l in range(K)] + [cand2_i[l, :] for l in range(K)]
        accv = neg
        acci = jnp.zeros((LANES,), jnp.int32)
        for r in range(K):
            m = vs[0]
            for v in vs[1:]:
                m = jnp.maximum(m, v)
            gmax = jnp.full((LANES,), jnp.max(m))
            big = jnp.full((LANES,), I32_BIG, jnp.int32)
            cand = [jnp.where(vs[t] == gmax, ids[t], big) for t in range(2 * K)]
            mn = cand[0]
            for cnd in cand[1:]:
                mn = jnp.minimum(mn, cnd)
            gidx = jnp.full((LANES,), jnp.min(mn))
            vs = [jnp.where((vs[t] == gmax) & (ids[t] == gidx), neg, vs[t])
                  for t in range(2 * K)]
            accv = jnp.where(iota == r, gmax, accv)
            acci = jnp.where(iota == r, gidx, acci)
        beams = jnp.zeros((LANES,), jnp.int32)
        for t in range(1, BEAM):
            beams = beams + jnp.where(acci >= t * VOCAB, 1, 0)
        toks = acci - beams * VOCAB
        rowf[...] = accv
        rowt[...] = toks
        rowb[...] = beams
        pltpu.sync_copy(rowf.at[pl.ds(0, K)], out_s.at[batch])
        pltpu.sync_copy(rowt.at[pl.ds(0, K)], out_t.at[batch])
        pltpu.sync_copy(rowb.at[pl.ds(0, K)], out_b.at[batch])


_sc_call = functools.partial(
    pl.kernel,
    out_type=[
        jax.ShapeDtypeStruct((BSZ, BEAM), jnp.float32),
        jax.ShapeDtypeStruct((BSZ, BEAM), jnp.int32),
        jax.ShapeDtypeStruct((BSZ, BEAM), jnp.int32),
    ],
    mesh=plsc.VectorSubcoreMesh(core_axis_name="c", subcore_axis_name="s"),
    scratch_types=[
        pltpu.VMEM((BSZ * LANES,), jnp.int32),
        pltpu.VMEM((BSZ * BEAM * LANES,), jnp.float32),
        pltpu.VMEM((HALF, WINDOW), jnp.float32),
        pltpu.VMEM((HALF * (NBLK + 1) * LANES,), jnp.float32),
        pltpu.VMEM((K, LANES), jnp.float32),
        pltpu.VMEM((K, LANES), jnp.int32),
        pltpu.VMEM((K, LANES), jnp.float32),
        pltpu.VMEM((K, LANES), jnp.int32),
        pltpu.VMEM((LANES,), jnp.float32),
        pltpu.VMEM((LANES,), jnp.int32),
        pltpu.VMEM((LANES,), jnp.int32),
        pltpu.VMEM_SHARED((LANES, K, LANES), jnp.float32),
        pltpu.VMEM_SHARED((LANES, K, LANES), jnp.int32),
        pltpu.SemaphoreType.DMA,
    ],
    compiler_params=pltpu.CompilerParams(
        use_tc_tiling_on_sc=False, needs_layout_passes=False),
)(_sc_body)


def kernel(step, lprobs, scores, prev_output_tokens, original_batch_idxs):
    starts = (original_batch_idxs.astype(jnp.int32) * 1000) % VOCAB
    starts = jnp.repeat(starts, LANES)
    bias = jnp.repeat(jnp.take(scores, step - 1, axis=2).reshape(-1), LANES)
    scores_buf, indices_buf, beams_buf = _sc_call(starts, bias, lprobs)
    return scores_buf, indices_buf, beams_buf


# speedup vs baseline: 1.1999x; 1.0116x over previous
import functools

import jax
import jax.numpy as jnp
from jax import lax
from jax.experimental import pallas as pl
from jax.experimental.pallas import tpu as pltpu
from jax.experimental.pallas import tpu_sc as plsc

VOCAB = 100000
WINDOW = 5000
BSZ = 16
BEAM = 8
K = 8
HALF = BEAM // 2
LANES = 16
NVEC = WINDOW // LANES
BLK = 8
BLKW = BLK * LANES
NBLK = NVEC // BLK
STEPS = 4
NEG_INF = float("-inf")
I32_BIG = 2**31 - 1


def _insert_topk(R, RI, x, xi):
    for lvl in range(K):
        swap = x > R[lvl]
        R[lvl], x = jnp.where(swap, x, R[lvl]), jnp.where(swap, R[lvl], x)
        RI[lvl], xi = jnp.where(swap, xi, RI[lvl]), jnp.where(swap, RI[lvl], xi)
    return R, RI


def _sc_body(step_hbm, orig_hbm, scores_hbm, lprobs_hbm, out_s, out_t, out_b,
             step_v, orig_v, scores_v, lbuf, blkmax_v, cand_v, cand_i,
             cand2_v, cand2_i, rowf, rowt, rowb, sh_v, sh_i, sem):
    c = lax.axis_index("c")
    s = lax.axis_index("s")
    batch = c * 8 + s // 2
    half = s % 2

    pltpu.sync_copy(step_hbm, step_v)
    pltpu.sync_copy(orig_hbm, orig_v)
    pltpu.sync_copy(scores_hbm, scores_v)
    iota = lax.iota(jnp.int32, LANES)
    neg = jnp.full((LANES,), NEG_INF, jnp.float32)

    bsplat = jnp.full((LANES,), batch, jnp.int32)
    ob = plsc.load_gather(orig_v, [bsplat])
    start_vec = (ob * 1000) % VOCAB
    start = pl.multiple_of(start_vec[0], 8)

    stepm1 = step_v[pl.ds(0, LANES)] - 1

    cps = []
    for j in range(HALF):
        jg = half * HALF + j
        cps.append(pltpu.async_copy(
            lprobs_hbm.at[batch, jg, pl.ds(start, WINDOW)], lbuf.at[j], sem))
    for cp in cps:
        cp.wait()

    biases = []
    M = neg
    for j in range(HALF):
        jg = half * HALF + j
        bidx = jnp.full((LANES,), (batch * BEAM + jg) * STEPS, jnp.int32) + stepm1
        bias = plsc.load_gather(scores_v, [bidx])
        biases.append(bias)

        def bodyA(b, M, _j=j, _bias=bias):
            m = lbuf[_j, pl.ds(b * BLKW, LANES)]
            for u in range(1, BLK):
                m = jnp.maximum(m, lbuf[_j, pl.ds(b * BLKW + u * LANES, LANES)])
            m = m + _bias
            blkmax_v[pl.ds((_j * (NBLK + 1) + b) * LANES, LANES)] = m
            return jnp.maximum(M, m)

        M = lax.fori_loop(0, NBLK, bodyA, M)
        m = lbuf[j, pl.ds(WINDOW - LANES, LANES)] + bias
        M = jnp.maximum(M, m)

    ms, _ = plsc.sort_key_val(M, M)
    t = jnp.full((LANES,), ms[LANES - K])

    carry = tuple([neg] * K + [jnp.zeros((LANES,), jnp.int32)] * K)
    for j in range(HALF):
        jg = half * HALF + j
        bias = biases[j]
        base = jg * VOCAB + start

        def bodyB(b, car, _j=j, _bias=bias, _base=base):
            bm = blkmax_v[pl.ds((_j * (NBLK + 1) + b) * LANES, LANES)]
            hit = plsc.all_reduce_population_count(bm >= t)[0] > 0

            def do(car):
                R = list(car[:K])
                RI = list(car[K:])
                for u in range(BLK):
                    x = lbuf[_j, pl.ds(b * BLKW + u * LANES, LANES)] + _bias
                    xi = jnp.full((LANES,), _base + b * BLKW + u * LANES,
                                  jnp.int32) + iota
                    R, RI = _insert_topk(R, RI, x, xi)
                return tuple(R + RI)

            return lax.cond(hit, do, lambda car: car, car)

        carry = lax.fori_loop(0, NBLK, bodyB, carry)
        R = list(carry[:K])
        RI = list(carry[K:])
        x = lbuf[j, pl.ds(WINDOW - LANES, LANES)] + bias
        xi = jnp.full((LANES,), base + WINDOW - LANES, jnp.int32) + iota
        R, RI = _insert_topk(R, RI, x, xi)
        carry = tuple(R + RI)

    R = list(carry[:K])
    RI = list(carry[K:])
    for lvl in range(K):
        cand_v[lvl, :] = R[lvl]
        cand_i[lvl, :] = RI[lvl]

    pltpu.sync_copy(cand_v, sh_v.at[s])
    pltpu.sync_copy(cand_i, sh_i.at[s])
    plsc.subcore_barrier()

    @pl.when(half == 0)
    def _merge():
        pltpu.sync_copy(sh_v.at[s + 1], cand2_v)
        pltpu.sync_copy(sh_i.at[s + 1], cand2_i)
        vs = [cand_v[l, :] for l in range(K)] + [cand2_v[l, :] for l in range(K)]
        ids = [cand_i[l, :] for l in range(K)] + [cand2_i[l, :] for l in range(K)]
        accv = neg
        acci = jnp.zeros((LANES,), jnp.int32)
        for r in range(K):
            m = vs[0]
            for v in vs[1:]:
                m = jnp.maximum(m, v)
            gmax = jnp.full((LANES,), jnp.max(m))
            big = jnp.full((LANES,), I32_BIG, jnp.int32)
            cand = [jnp.where(vs[t] == gmax, ids[t], big) for t in range(2 * K)]
            mn = cand[0]
            for cnd in cand[1:]:
                mn = jnp.minimum(mn, cnd)
            gidx = jnp.full((LANES,), jnp.min(mn))
            vs = [jnp.where((vs[t] == gmax) & (ids[t] == gidx), neg, vs[t])
                  for t in range(2 * K)]
            accv = jnp.where(iota == r, gmax, accv)
            acci = jnp.where(iota == r, gidx, acci)
        beams = jnp.zeros((LANES,), jnp.int32)
        for t in range(1, BEAM):
            beams = beams + jnp.where(acci >= t * VOCAB, 1, 0)
        toks = acci - beams * VOCAB
        rowf[...] = accv
        rowt[...] = toks
        rowb[...] = beams
        pltpu.sync_copy(rowf.at[pl.ds(0, K)], out_s.at[batch])
        pltpu.sync_copy(rowt.at[pl.ds(0, K)], out_t.at[batch])
        pltpu.sync_copy(rowb.at[pl.ds(0, K)], out_b.at[batch])


_sc_call = functools.partial(
    pl.kernel,
    out_type=[
        jax.ShapeDtypeStruct((BSZ, BEAM), jnp.float32),
        jax.ShapeDtypeStruct((BSZ, BEAM), jnp.int32),
        jax.ShapeDtypeStruct((BSZ, BEAM), jnp.int32),
    ],
    mesh=plsc.VectorSubcoreMesh(core_axis_name="c", subcore_axis_name="s"),
    scratch_types=[
        pltpu.VMEM((LANES,), jnp.int32),
        pltpu.VMEM((BSZ,), jnp.int32),
        pltpu.VMEM((BSZ * BEAM * STEPS,), jnp.float32),
        pltpu.VMEM((HALF, WINDOW), jnp.float32),
        pltpu.VMEM((HALF * (NBLK + 1) * LANES,), jnp.float32),
        pltpu.VMEM((K, LANES), jnp.float32),
        pltpu.VMEM((K, LANES), jnp.int32),
        pltpu.VMEM((K, LANES), jnp.float32),
        pltpu.VMEM((K, LANES), jnp.int32),
        pltpu.VMEM((LANES,), jnp.float32),
        pltpu.VMEM((LANES,), jnp.int32),
        pltpu.VMEM((LANES,), jnp.int32),
        pltpu.VMEM_SHARED((LANES, K, LANES), jnp.float32),
        pltpu.VMEM_SHARED((LANES, K, LANES), jnp.int32),
        pltpu.SemaphoreType.DMA,
    ],
    compiler_params=pltpu.CompilerParams(
        use_tc_tiling_on_sc=False, needs_layout_passes=False),
)(_sc_body)


def kernel(step, lprobs, scores, prev_output_tokens, original_batch_idxs):
    step16 = jnp.broadcast_to(jnp.asarray(step, jnp.int32), (LANES,))
    scores_buf, indices_buf, beams_buf = _sc_call(
        step16, original_batch_idxs.astype(jnp.int32), scores.reshape(-1),
        lprobs)
    return scores_buf, indices_buf, beams_buf
